# baseline (speedup 1.0000x reference)
import functools

import jax
import jax.numpy as jnp
from jax import lax
from jax.experimental import pallas as pl
from jax.experimental.pallas import tpu as pltpu
from jax.experimental.pallas import tpu_sc as plsc

N = 50000
M = 16
E = N * M
B = 256
AF = 64
NF = 41
OAF = 92
ED = 128
NCONV = 3
CW = 2 * AF

NB = 200
EBLK = NB * M
GRID = N // NB

NW = 32
CH = 896
PW = 25088
PAD_E = NW * PW
NCH = PW // CH
NCH2 = NCH // 2


def _sc_gather(table, idx1d):
    mesh = plsc.VectorSubcoreMesh(core_axis_name="c", subcore_axis_name="s")

    @functools.partial(
        pl.kernel,
        out_type=jax.ShapeDtypeStruct((PAD_E, CW), jnp.bfloat16),
        mesh=mesh,
        scratch_types=[
            pltpu.VMEM((CH,), jnp.int32),
            pltpu.VMEM((CH,), jnp.int32),
            pltpu.VMEM((CH, CW), jnp.bfloat16),
            pltpu.VMEM((CH, CW), jnp.bfloat16),
            pltpu.SemaphoreType.DMA,
            pltpu.SemaphoreType.DMA,
        ],
        compiler_params=pltpu.CompilerParams(use_tc_tiling_on_sc=False),
    )
    def k(table_hbm, idx_hbm, out_hbm, idx_a, idx_b, rows_a, rows_b,
          sem_a, sem_b):
        c = lax.axis_index("c")
        s = lax.axis_index("s")
        wid = s * 2 + c
        base = wid * PW

        pltpu.sync_copy(idx_hbm.at[pl.ds(base, CH)], idx_a)
        pltpu.make_async_copy(table_hbm.at[idx_a], rows_a, sem_a).start()

        def body(j2, carry):
            off = base + (2 * j2) * CH
            pltpu.sync_copy(idx_hbm.at[pl.ds(off + CH, CH)], idx_b)
            pltpu.make_async_copy(table_hbm.at[idx_b], rows_b, sem_b).start()
            pltpu.make_async_copy(table_hbm.at[idx_a], rows_a, sem_a).wait()
            pltpu.sync_copy(rows_a, out_hbm.at[pl.ds(off, CH)])

            @pl.when(j2 + 1 < NCH2)
            def _():
                pltpu.sync_copy(idx_hbm.at[pl.ds(off + 2 * CH, CH)], idx_a)
                pltpu.make_async_copy(table_hbm.at[idx_a], rows_a,
                                      sem_a).start()

            pltpu.make_async_copy(table_hbm.at[idx_b], rows_b, sem_b).wait()
            pltpu.sync_copy(rows_b, out_hbm.at[pl.ds(off + CH, CH)])
            return carry

        lax.fori_loop(0, NCH2, body, 0)

    return k(table, idx1d)


def _atom0_body(x_ref, w_ref, b_ref, wn_ref, o_ref, a_ref):
    atom = (
        jnp.dot(x_ref[...], w_ref[...], preferred_element_type=jnp.float32)
        + b_ref[...]
    )
    o_ref[...] = atom
    a_ref[...] = jnp.dot(
        atom, wn_ref[...], preferred_element_type=jnp.float32
    ).astype(jnp.bfloat16)


def _edge_g(atom_ref, gat_ref, fea_ref, ws_ref, we_ref, b_ref):
    s = (
        jnp.dot(atom_ref[...], ws_ref[...], preferred_element_type=jnp.float32)
        + b_ref[...]
    )
    g = jnp.broadcast_to(s[:, None, :], (NB, M, CW)).reshape(EBLK, CW)
    g = g + gat_ref[...].astype(jnp.float32)
    g = g + jnp.dot(fea_ref[...], we_ref[...], preferred_element_type=jnp.float32)
    return g


def _passA_body(atom_ref, gat_ref, fea_ref, ws_ref, we_ref, b_ref,
                stat_ref):
    i = pl.program_id(0)
    g = _edge_g(atom_ref, gat_ref, fea_ref, ws_ref, we_ref, b_ref)

    @pl.when(i == 0)
    def _():
        stat_ref[...] = jnp.zeros_like(stat_ref)

    stat_ref[0:1, :] += jnp.sum(g, axis=0)[None]
    stat_ref[1:2, :] += jnp.sum(g * g, axis=0)[None]


def _passB_body(atom_ref, gat_ref, fea_ref, ws_ref, we_ref, b_ref,
                a1_ref, c1_ref, ns_ref, stat_ref):
    i = pl.program_id(0)
    g = _edge_g(atom_ref, gat_ref, fea_ref, ws_ref, we_ref, b_ref)
    gh = g * a1_ref[...] + c1_ref[...]
    filt = jax.nn.sigmoid(gh[:, :AF])
    core = jax.nn.softplus(gh[:, AF:])
    p = (filt * core).reshape(NB, M, AF).sum(axis=1)
    ns_ref[...] = p

    @pl.when(i == 0)
    def _():
        stat_ref[...] = jnp.zeros_like(stat_ref)

    stat_ref[0:1, :] += jnp.sum(p, axis=0)[None]
    stat_ref[1:2, :] += jnp.sum(p * p, axis=0)[None]


def _passC_body(atom_ref, ns_ref, a2_ref, c2_ref, o_ref):
    o_ref[...] = jax.nn.softplus(
        atom_ref[...] + ns_ref[...] * a2_ref[...] + c2_ref[...]
    )


def _passCA_body(atom_ref, ns_ref, a2_ref, c2_ref, wn_ref, o_ref, a_ref):
    atom = jax.nn.softplus(
        atom_ref[...] + ns_ref[...] * a2_ref[...] + c2_ref[...]
    )
    o_ref[...] = atom
    a_ref[...] = jnp.dot(
        atom, wn_ref[...], preferred_element_type=jnp.float32
    ).astype(jnp.bfloat16)


def _pool_body(atom_ref, batch_ref, sum_ref, cnt_ref):
    i = pl.program_id(0)
    bb = batch_ref[0, 0, :]
    seg = lax.broadcasted_iota(jnp.int32, (NB, B), 1)
    onehot = (bb[:, None] == seg).astype(jnp.float32)

    @pl.when(i == 0)
    def _():
        sum_ref[...] = jnp.zeros_like(sum_ref)
        cnt_ref[...] = jnp.zeros_like(cnt_ref)

    sum_ref[...] += lax.dot_general(
        onehot, atom_ref[...], (((0,), (0,)), ((), ())),
        preferred_element_type=jnp.float32,
    )
    cnt_ref[0:1, :] += jnp.sum(onehot, axis=0)[None]


def _head_body(sum_ref, cnt_ref, wfc_ref, bfc_ref, wout_ref, bout_ref, o_ref):
    cnt = jnp.maximum(cnt_ref[0, :], 1.0)
    mean = sum_ref[...] / cnt[:, None]
    h = jax.nn.softplus(mean)
    h = jnp.dot(h, wfc_ref[...], preferred_element_type=jnp.float32) + bfc_ref[...]
    h = jax.nn.softplus(h)
    o_ref[...] = (
        jnp.dot(h, wout_ref[...], preferred_element_type=jnp.float32)
        + bout_ref[...]
    )


def _full(shape):
    return pl.BlockSpec(shape, lambda *a: tuple(0 for _ in shape))


def _atom0(x, w, b, wn):
    return pl.pallas_call(
        _atom0_body,
        grid=(GRID,),
        in_specs=[
            pl.BlockSpec((NB, OAF), lambda i: (i, 0)),
            _full((OAF, AF)),
            _full((1, AF)),
            _full((AF, CW)),
        ],
        out_specs=[
            pl.BlockSpec((NB, AF), lambda i: (i, 0)),
            pl.BlockSpec((NB, CW), lambda i: (i, 0)),
        ],
        out_shape=[
            jax.ShapeDtypeStruct((N, AF), jnp.float32),
            jax.ShapeDtypeStruct((N, CW), jnp.bfloat16),
        ],
    )(x, w, b, wn)


_EDGE_SPECS = [
    pl.BlockSpec((NB, AF), lambda i: (i, 0)),
    pl.BlockSpec((EBLK, CW), lambda i: (i, 0)),
    pl.BlockSpec((EBLK, NF), lambda i: (i, 0)),
    _full((AF, CW)),
    _full((NF, CW)),
    _full((1, CW)),
]


def _passA(atom, gat, fea, ws, we, b):
    return pl.pallas_call(
        _passA_body,
        grid=(GRID,),
        in_specs=_EDGE_SPECS,
        out_specs=_full((8, CW)),
        out_shape=jax.ShapeDtypeStruct((8, CW), jnp.float32),
    )(atom, gat, fea, ws, we, b)


def _passB(atom, gat, fea, ws, we, b, a1, c1):
    return pl.pallas_call(
        _passB_body,
        grid=(GRID,),
        in_specs=_EDGE_SPECS + [_full((1, CW)), _full((1, CW))],
        out_specs=[
            pl.BlockSpec((NB, AF), lambda i: (i, 0)),
            _full((8, AF)),
        ],
        out_shape=[
            jax.ShapeDtypeStruct((N, AF), jnp.float32),
            jax.ShapeDtypeStruct((8, AF), jnp.float32),
        ],
    )(atom, gat, fea, ws, we, b, a1, c1)


def _passC(atom, ns, a2, c2):
    return pl.pallas_call(
        _passC_body,
        grid=(GRID,),
        in_specs=[
            pl.BlockSpec((NB, AF), lambda i: (i, 0)),
            pl.BlockSpec((NB, AF), lambda i: (i, 0)),
            _full((1, AF)),
            _full((1, AF)),
        ],
        out_specs=pl.BlockSpec((NB, AF), lambda i: (i, 0)),
        out_shape=jax.ShapeDtypeStruct((N, AF), jnp.float32),
    )(atom, ns, a2, c2)


def _passCA(atom, ns, a2, c2, wn):
    return pl.pallas_call(
        _passCA_body,
        grid=(GRID,),
        in_specs=[
            pl.BlockSpec((NB, AF), lambda i: (i, 0)),
            pl.BlockSpec((NB, AF), lambda i: (i, 0)),
            _full((1, AF)),
            _full((1, AF)),
            _full((AF, CW)),
        ],
        out_specs=[
            pl.BlockSpec((NB, AF), lambda i: (i, 0)),
            pl.BlockSpec((NB, CW), lambda i: (i, 0)),
        ],
        out_shape=[
            jax.ShapeDtypeStruct((N, AF), jnp.float32),
            jax.ShapeDtypeStruct((N, CW), jnp.bfloat16),
        ],
    )(atom, ns, a2, c2, wn)


def _pool(atom, batch3):
    return pl.pallas_call(
        _pool_body,
        grid=(GRID,),
        in_specs=[
            pl.BlockSpec((NB, AF), lambda i: (i, 0)),
            pl.BlockSpec((1, 1, NB), lambda i: (i, 0, 0)),
        ],
        out_specs=[_full((B, AF)), _full((8, B))],
        out_shape=[
            jax.ShapeDtypeStruct((B, AF), jnp.float32),
            jax.ShapeDtypeStruct((8, B), jnp.float32),
        ],
    )(atom, batch3)


def _head(sums, cnts, wfc, bfc, wout, bout):
    return pl.pallas_call(
        _head_body,
        in_specs=[
            _full((B, AF)),
            _full((8, B)),
            _full((AF, ED)),
            _full((1, ED)),
            _full((ED, 1)),
            _full((1, 1)),
        ],
        out_specs=_full((B, 1)),
        out_shape=jax.ShapeDtypeStruct((B, 1), jnp.float32),
    )(sums, cnts, wfc, bfc, wout, bout)


def kernel(x, edge_attr, edge_index, batch, W_in, b_in, convW, convb,
           bn1_g, bn1_b, bn2_g, bn2_b, W_fc, b_fc, W_out, b_out):
    idx = edge_index[1].astype(jnp.int32)
    idx1d = jnp.pad(idx, (0, PAD_E - E))

    atom, a_tbl = _atom0(x, W_in, b_in.reshape(1, AF), convW[0, AF:2 * AF])
    for i in range(NCONV):
        gathered = _sc_gather(a_tbl, idx1d)
        ws = convW[i, :AF]
        we = convW[i, 2 * AF:]
        bb = convb[i].reshape(1, CW)

        stat1 = _passA(atom, gathered, edge_attr, ws, we, bb)
        m1 = stat1[0] / E
        v1 = stat1[1] / E - m1 * m1
        a1 = bn1_g[i] / jnp.sqrt(v1 + 1e-5)
        c1 = bn1_b[i] - m1 * a1

        ns, stat2 = _passB(atom, gathered, edge_attr, ws, we, bb,
                           a1.reshape(1, CW), c1.reshape(1, CW))
        m2 = stat2[0] / N
        v2 = stat2[1] / N - m2 * m2
        a2 = bn2_g[i] / jnp.sqrt(v2 + 1e-5)
        c2 = bn2_b[i] - m2 * a2

        if i + 1 < NCONV:
            atom, a_tbl = _passCA(atom, ns, a2.reshape(1, AF),
                                  c2.reshape(1, AF), convW[i + 1, AF:2 * AF])
        else:
            atom = _passC(atom, ns, a2.reshape(1, AF), c2.reshape(1, AF))

    batch3 = batch.astype(jnp.int32).reshape(GRID, 1, NB)
    sums, cnts = _pool(atom, batch3)
    return _head(sums, cnts, W_fc, b_fc.reshape(1, ED), W_out,
                 b_out.reshape(1, 1))


# speedup vs baseline: 1.4103x; 1.4103x over previous
import functools

import jax
import jax.numpy as jnp
from jax import lax
from jax.experimental import pallas as pl
from jax.experimental.pallas import tpu as pltpu
from jax.experimental.pallas import tpu_sc as plsc

N = 50000
M = 16
E = N * M
B = 256
AF = 64
NF = 41
OAF = 92
ED = 128
NCONV = 3
CW = 2 * AF

NB = 200
EBLK = NB * M
GRID = N // NB

NW = 32
CH = 448
PW = 12544
PAD_H = NW * PW
NCH = PW // CH
NCH2 = NCH // 2
EH = E // 2
GRIDH = GRID // 2


def _sc_gather(table, idx1d):
    mesh = plsc.VectorSubcoreMesh(core_axis_name="c", subcore_axis_name="s")

    @functools.partial(
        pl.kernel,
        out_type=jax.ShapeDtypeStruct((PAD_H, CW), jnp.float32),
        mesh=mesh,
        scratch_types=[
            pltpu.VMEM((CH,), jnp.int32),
            pltpu.VMEM((CH,), jnp.int32),
            pltpu.VMEM((CH, CW), jnp.float32),
            pltpu.VMEM((CH, CW), jnp.float32),
            pltpu.SemaphoreType.DMA,
            pltpu.SemaphoreType.DMA,
        ],
        compiler_params=pltpu.CompilerParams(use_tc_tiling_on_sc=False),
    )
    def k(table_hbm, idx_hbm, out_hbm, idx_a, idx_b, rows_a, rows_b,
          sem_a, sem_b):
        c = lax.axis_index("c")
        s = lax.axis_index("s")
        wid = s * 2 + c
        base = wid * PW

        pltpu.sync_copy(idx_hbm.at[pl.ds(base, CH)], idx_a)
        pltpu.make_async_copy(table_hbm.at[idx_a], rows_a, sem_a).start()

        def body(j2, carry):
            off = base + (2 * j2) * CH
            pltpu.sync_copy(idx_hbm.at[pl.ds(off + CH, CH)], idx_b)
            pltpu.make_async_copy(table_hbm.at[idx_b], rows_b, sem_b).start()
            pltpu.make_async_copy(table_hbm.at[idx_a], rows_a, sem_a).wait()
            pltpu.sync_copy(rows_a, out_hbm.at[pl.ds(off, CH)])

            @pl.when(j2 + 1 < NCH2)
            def _():
                pltpu.sync_copy(idx_hbm.at[pl.ds(off + 2 * CH, CH)], idx_a)
                pltpu.make_async_copy(table_hbm.at[idx_a], rows_a,
                                      sem_a).start()

            pltpu.make_async_copy(table_hbm.at[idx_b], rows_b, sem_b).wait()
            pltpu.sync_copy(rows_b, out_hbm.at[pl.ds(off + CH, CH)])
            return carry

        lax.fori_loop(0, NCH2, body, 0)

    return k(table, idx1d)


def _atom0_body(x_ref, w_ref, b_ref, wn_ref, o_ref, a_ref):
    atom = (
        jnp.dot(x_ref[...], w_ref[...], preferred_element_type=jnp.float32)
        + b_ref[...]
    )
    o_ref[...] = atom
    a_ref[...] = jnp.dot(atom, wn_ref[...], preferred_element_type=jnp.float32)


def _edge_g(atom_ref, gat_ref, fea_ref, ws_ref, we_ref, b_ref):
    s = (
        jnp.dot(atom_ref[...], ws_ref[...], preferred_element_type=jnp.float32)
        + b_ref[...]
    )
    g = jnp.broadcast_to(s[:, None, :], (NB, M, CW)).reshape(EBLK, CW)
    g = g + gat_ref[...]
    g = g + jnp.dot(fea_ref[...], we_ref[...], preferred_element_type=jnp.float32)
    return g


def _passA_body(atom_ref, gat_ref, fea_ref, ws_ref, we_ref, b_ref,
                stat_ref):
    i = pl.program_id(0)
    g = _edge_g(atom_ref, gat_ref, fea_ref, ws_ref, we_ref, b_ref)

    @pl.when(i == 0)
    def _():
        stat_ref[...] = jnp.zeros_like(stat_ref)

    stat_ref[0:1, :] += jnp.sum(g, axis=0)[None]
    stat_ref[1:2, :] += jnp.sum(g * g, axis=0)[None]


def _passB_body(atom_ref, gat_ref, fea_ref, ws_ref, we_ref, b_ref,
                a1_ref, c1_ref, *refs):
    ns_ref, stat_ref = refs[-2], refs[-1]
    i = pl.program_id(0)
    g = _edge_g(atom_ref, gat_ref, fea_ref, ws_ref, we_ref, b_ref)
    gh = g * a1_ref[...] + c1_ref[...]
    filt = jax.nn.sigmoid(gh[:, :AF])
    core = jax.nn.softplus(gh[:, AF:])
    p = (filt * core).reshape(NB, M, AF).sum(axis=1)
    ns_ref[...] = p

    @pl.when(i == 0)
    def _():
        stat_ref[...] = jnp.zeros_like(stat_ref)

    stat_ref[0:1, :] += jnp.sum(p, axis=0)[None]
    stat_ref[1:2, :] += jnp.sum(p * p, axis=0)[None]


def _passC_body(atom_ref, ns_ref, a2_ref, c2_ref, o_ref):
    o_ref[...] = jax.nn.softplus(
        atom_ref[...] + ns_ref[...] * a2_ref[...] + c2_ref[...]
    )


def _passCA_body(atom_ref, ns_ref, a2_ref, c2_ref, wn_ref, o_ref, a_ref):
    atom = jax.nn.softplus(
        atom_ref[...] + ns_ref[...] * a2_ref[...] + c2_ref[...]
    )
    o_ref[...] = atom
    a_ref[...] = jnp.dot(atom, wn_ref[...], preferred_element_type=jnp.float32)


def _pool_body(atom_ref, batch_ref, sum_ref, cnt_ref):
    i = pl.program_id(0)
    bb = batch_ref[0, 0, :]
    seg = lax.broadcasted_iota(jnp.int32, (NB, B), 1)
    onehot = (bb[:, None] == seg).astype(jnp.float32)

    @pl.when(i == 0)
    def _():
        sum_ref[...] = jnp.zeros_like(sum_ref)
        cnt_ref[...] = jnp.zeros_like(cnt_ref)

    sum_ref[...] += lax.dot_general(
        onehot, atom_ref[...], (((0,), (0,)), ((), ())),
        preferred_element_type=jnp.float32,
    )
    cnt_ref[0:1, :] += jnp.sum(onehot, axis=0)[None]


def _head_body(sum_ref, cnt_ref, wfc_ref, bfc_ref, wout_ref, bout_ref, o_ref):
    cnt = jnp.maximum(cnt_ref[0, :], 1.0)
    mean = sum_ref[...] / cnt[:, None]
    h = jax.nn.softplus(mean)
    h = jnp.dot(h, wfc_ref[...], preferred_element_type=jnp.float32) + bfc_ref[...]
    h = jax.nn.softplus(h)
    o_ref[...] = (
        jnp.dot(h, wout_ref[...], preferred_element_type=jnp.float32)
        + bout_ref[...]
    )


def _full(shape):
    return pl.BlockSpec(shape, lambda *a: tuple(0 for _ in shape))


def _atom0(x, w, b, wn):
    return pl.pallas_call(
        _atom0_body,
        grid=(GRID,),
        in_specs=[
            pl.BlockSpec((NB, OAF), lambda i: (i, 0)),
            _full((OAF, AF)),
            _full((1, AF)),
            _full((AF, CW)),
        ],
        out_specs=[
            pl.BlockSpec((NB, AF), lambda i: (i, 0)),
            pl.BlockSpec((NB, CW), lambda i: (i, 0)),
        ],
        out_shape=[
            jax.ShapeDtypeStruct((N, AF), jnp.float32),
            jax.ShapeDtypeStruct((N, CW), jnp.float32),
        ],
    )(x, w, b, wn)


def _half_edge_specs(off):
    return [
        pl.BlockSpec((NB, AF), lambda i: (i + off, 0)),
        pl.BlockSpec((EBLK, CW), lambda i: (i, 0)),
        pl.BlockSpec((EBLK, NF), lambda i: (i + off, 0)),
        _full((AF, CW)),
        _full((NF, CW)),
        _full((1, CW)),
    ]


def _passA(atom, gat, fea, ws, we, b, off):
    return pl.pallas_call(
        _passA_body,
        grid=(GRIDH,),
        in_specs=_half_edge_specs(off),
        out_specs=_full((8, CW)),
        out_shape=jax.ShapeDtypeStruct((8, CW), jnp.float32),
    )(atom, gat, fea, ws, we, b)


def _passB(atom, gat, fea, ws, we, b, a1, c1, off, ns_prev=None):
    ins = [atom, gat, fea, ws, we, b, a1, c1]
    in_specs = _half_edge_specs(off) + [_full((1, CW)), _full((1, CW))]
    aliases = {}
    if ns_prev is not None:
        ins.append(ns_prev)
        in_specs.append(pl.BlockSpec((NB, AF), lambda i: (i + off, 0)))
        aliases = {8: 0}
    return pl.pallas_call(
        _passB_body,
        grid=(GRIDH,),
        in_specs=in_specs,
        out_specs=[
            pl.BlockSpec((NB, AF), lambda i: (i + off, 0)),
            _full((8, AF)),
        ],
        out_shape=[
            jax.ShapeDtypeStruct((N, AF), jnp.float32),
            jax.ShapeDtypeStruct((8, AF), jnp.float32),
        ],
        input_output_aliases=aliases,
    )(*ins)


def _passC(atom, ns, a2, c2):
    return pl.pallas_call(
        _passC_body,
        grid=(GRID,),
        in_specs=[
            pl.BlockSpec((NB, AF), lambda i: (i, 0)),
            pl.BlockSpec((NB, AF), lambda i: (i, 0)),
            _full((1, AF)),
            _full((1, AF)),
        ],
        out_specs=pl.BlockSpec((NB, AF), lambda i: (i, 0)),
        out_shape=jax.ShapeDtypeStruct((N, AF), jnp.float32),
    )(atom, ns, a2, c2)


def _passCA(atom, ns, a2, c2, wn):
    return pl.pallas_call(
        _passCA_body,
        grid=(GRID,),
        in_specs=[
            pl.BlockSpec((NB, AF), lambda i: (i, 0)),
            pl.BlockSpec((NB, AF), lambda i: (i, 0)),
            _full((1, AF)),
            _full((1, AF)),
            _full((AF, CW)),
        ],
        out_specs=[
            pl.BlockSpec((NB, AF), lambda i: (i, 0)),
            pl.BlockSpec((NB, CW), lambda i: (i, 0)),
        ],
        out_shape=[
            jax.ShapeDtypeStruct((N, AF), jnp.float32),
            jax.ShapeDtypeStruct((N, CW), jnp.float32),
        ],
    )(atom, ns, a2, c2, wn)


def _pool(atom, batch3):
    return pl.pallas_call(
        _pool_body,
        grid=(GRID,),
        in_specs=[
            pl.BlockSpec((NB, AF), lambda i: (i, 0)),
            pl.BlockSpec((1, 1, NB), lambda i: (i, 0, 0)),
        ],
        out_specs=[_full((B, AF)), _full((8, B))],
        out_shape=[
            jax.ShapeDtypeStruct((B, AF), jnp.float32),
            jax.ShapeDtypeStruct((8, B), jnp.float32),
        ],
    )(atom, batch3)


def _head(sums, cnts, wfc, bfc, wout, bout):
    return pl.pallas_call(
        _head_body,
        in_specs=[
            _full((B, AF)),
            _full((8, B)),
            _full((AF, ED)),
            _full((1, ED)),
            _full((ED, 1)),
            _full((1, 1)),
        ],
        out_specs=_full((B, 1)),
        out_shape=jax.ShapeDtypeStruct((B, 1), jnp.float32),
    )(sums, cnts, wfc, bfc, wout, bout)


def kernel(x, edge_attr, edge_index, batch, W_in, b_in, convW, convb,
           bn1_g, bn1_b, bn2_g, bn2_b, W_fc, b_fc, W_out, b_out):
    idx = edge_index[1].astype(jnp.int32)
    idx_h0 = jnp.pad(idx[:EH], (0, PAD_H - EH))
    idx_h1 = jnp.pad(idx[EH:], (0, PAD_H - EH))

    atom, a_tbl = _atom0(x, W_in, b_in.reshape(1, AF), convW[0, AF:2 * AF])
    for i in range(NCONV):
        gat0 = _sc_gather(a_tbl, idx_h0)
        gat1 = _sc_gather(a_tbl, idx_h1)
        ws = convW[i, :AF]
        we = convW[i, 2 * AF:]
        bb = convb[i].reshape(1, CW)

        stat1 = (_passA(atom, gat0, edge_attr, ws, we, bb, 0)
                 + _passA(atom, gat1, edge_attr, ws, we, bb, GRIDH))
        m1 = stat1[0] / E
        v1 = stat1[1] / E - m1 * m1
        a1 = bn1_g[i] / jnp.sqrt(v1 + 1e-5)
        c1 = bn1_b[i] - m1 * a1

        ns, statb0 = _passB(atom, gat0, edge_attr, ws, we, bb,
                            a1.reshape(1, CW), c1.reshape(1, CW), 0)
        ns, statb1 = _passB(atom, gat1, edge_attr, ws, we, bb,
                            a1.reshape(1, CW), c1.reshape(1, CW), GRIDH,
                            ns_prev=ns)
        stat2 = statb0 + statb1
        m2 = stat2[0] / N
        v2 = stat2[1] / N - m2 * m2
        a2 = bn2_g[i] / jnp.sqrt(v2 + 1e-5)
        c2 = bn2_b[i] - m2 * a2

        if i + 1 < NCONV:
            atom, a_tbl = _passCA(atom, ns, a2.reshape(1, AF),
                                  c2.reshape(1, AF), convW[i + 1, AF:2 * AF])
        else:
            atom = _passC(atom, ns, a2.reshape(1, AF), c2.reshape(1, AF))

    batch3 = batch.astype(jnp.int32).reshape(GRID, 1, NB)
    sums, cnts = _pool(atom, batch3)
    return _head(sums, cnts, W_fc, b_fc.reshape(1, ED), W_out,
                 b_out.reshape(1, 1))
